# baseline (speedup 1.0000x reference)
import jax
import jax.numpy as jnp
from jax.experimental import pallas as pl
from jax.experimental.pallas import tpu as pltpu

_BLOCK_ROWS = 512


def _fused_body(theta_ref, miu_ref, m0_ref, m1_ref, x0_ref, x1_ref, x2_ref,
                out0_ref, out1_ref, fused_ref):
    theta = theta_ref[0]
    miu = miu_ref[0]
    m0 = m0_ref[...]
    m1 = m1_ref[...]
    x0 = x0_ref[...]
    x1 = x1_ref[...]
    x2 = x2_ref[...]

    ge0 = m0 >= theta
    ge1 = m1 >= theta
    out0_ref[...] = jnp.where(ge0, x0, x1)
    out1_ref[...] = jnp.where(ge1, x1, x0)
    fused = jnp.where(m0 >= miu, x0, x2)
    fused_ref[...] = jnp.where(m1 >= miu, x1, fused)


def kernel(x0, x1, x2, mask0, mask1, mask_threshold_theta, mask_threshold_miu):
    B, N, C = x0.shape
    R = B * N
    x0f = x0.reshape(R, C)
    x1f = x1.reshape(R, C)
    x2f = x2.reshape(R, C)
    m0f = mask0.reshape(R, 1)
    m1f = mask1.reshape(R, 1)
    theta = jnp.asarray(mask_threshold_theta, jnp.float32).reshape(1)
    miu = jnp.asarray(mask_threshold_miu, jnp.float32).reshape(1)

    grid = (R // _BLOCK_ROWS,)
    row_block = pl.BlockSpec((_BLOCK_ROWS, C), lambda i: (i, 0))
    mask_block = pl.BlockSpec((_BLOCK_ROWS, 1), lambda i: (i, 0))
    scalar_spec = pl.BlockSpec(memory_space=pltpu.SMEM)

    out0, out1, fused = pl.pallas_call(
        _fused_body,
        grid=grid,
        in_specs=[scalar_spec, scalar_spec, mask_block, mask_block,
                  row_block, row_block, row_block],
        out_specs=[row_block, row_block, row_block],
        out_shape=[jax.ShapeDtypeStruct((R, C), jnp.float32)] * 3,
    )(theta, miu, m0f, m1f, x0f, x1f, x2f)

    return (out0.reshape(B, N, C), out1.reshape(B, N, C),
            fused.reshape(B, N, C))
